# Initial kernel scaffold; baseline (speedup 1.0000x reference)
import jax
import jax.numpy as jnp
from jax.experimental import pallas as pl


def kernel(class_logit, box_regression, proposal, image_w, image_h):
    raise NotImplementedError("write your pallas kernel here")


# speedup vs baseline: 10.9554x; 10.9554x over previous
import jax
import jax.numpy as jnp
import numpy as np
from jax import lax
from jax.experimental import pallas as pl
from jax.experimental.pallas import tpu as pltpu

_SCORE_THRESH = 0.05
_NMS_THRESH = 0.5
_NUM_DETECTIONS = 100
_MIN_SIZE = 1.0
_REG_WEIGHTS = (10.0, 10.0, 5.0, 5.0)
_LOG_MAX = float(np.log(1000.0 / 16.0))

_N = 1000
_NP = 1024
_C = 21
_CP = 32
_OUTP = 128
_NEG = -1e30


def _decode_clip(dx, dy, dw, dh, px1, py1, px2, py2, w_max, h_max):
    wx, wy, ww, wh = _REG_WEIGHTS
    dx = dx / wx
    dy = dy / wy
    dw = jnp.minimum(dw / ww, _LOG_MAX)
    dh = jnp.minimum(dh / wh, _LOG_MAX)
    w = px2 - px1
    h = py2 - py1
    cx = px1 + 0.5 * w
    cy = py1 + 0.5 * h
    pcx = dx * w + cx
    pcy = dy * h + cy
    pw = jnp.exp(dw) * w
    ph = jnp.exp(dh) * h
    x1 = jnp.clip(pcx - 0.5 * pw, 0.0, w_max)
    y1 = jnp.clip(pcy - 0.5 * ph, 0.0, h_max)
    x2 = jnp.clip(pcx + 0.5 * pw, 0.0, w_max)
    y2 = jnp.clip(pcy + 0.5 * ph, 0.0, h_max)
    return x1, y1, x2, y2


def _softmax_col(lg_slices, m, c):
    se = jnp.zeros_like(m)
    lc = jnp.zeros_like(m)
    for k in range(_C):
        se = se + jnp.exp(lg_slices[k] - m)
        sel = (c + 1 == k).astype(jnp.float32)
        lc = lc + lg_slices[k] * sel
    return jnp.exp(lc - m) / se


def _nms_body(wh_ref, lgt_ref, lgc_ref, dlt_ref, dlc_ref, prt_ref, prc_ref,
              out_ref):
    f32 = jnp.float32
    c = pl.program_id(0)
    w_max = wh_ref[0]
    h_max = wh_ref[1]

    lgt = lgt_ref[...]
    rows = [lgt[k:k + 1, :] for k in range(_C)]
    m_r = rows[0]
    for k in range(1, _C):
        m_r = jnp.maximum(m_r, rows[k])
    score_r = _softmax_col(rows, m_r, c)

    dlt = dlt_ref[0]
    prt = prt_ref[...]
    x1_r, y1_r, x2_r, y2_r = _decode_clip(
        dlt[0:1, :], dlt[1:2, :], dlt[2:3, :], dlt[3:4, :],
        prt[0:1, :], prt[1:2, :], prt[2:3, :], prt[3:4, :], w_max, h_max)
    j_r = lax.broadcasted_iota(jnp.int32, (1, _NP), 1)
    valid_r = (((x2_r - x1_r) >= _MIN_SIZE) & ((y2_r - y1_r) >= _MIN_SIZE)
               & (score_r >= _SCORE_THRESH) & (j_r < _N))
    s_r = score_r * valid_r.astype(f32)

    lgc = lgc_ref[...]
    cols = [lgc[:, k:k + 1] for k in range(_C)]
    m_c = cols[0]
    for k in range(1, _C):
        m_c = jnp.maximum(m_c, cols[k])
    score_c = _softmax_col(cols, m_c, c)

    dlc = dlc_ref[0]
    prc = prc_ref[...]
    x1_c, y1_c, x2_c, y2_c = _decode_clip(
        dlc[:, 0:1], dlc[:, 1:2], dlc[:, 2:3], dlc[:, 3:4],
        prc[:, 0:1], prc[:, 1:2], prc[:, 2:3], prc[:, 3:4], w_max, h_max)
    i_c = lax.broadcasted_iota(jnp.int32, (_NP, 1), 0)
    valid_c = (((x2_c - x1_c) >= _MIN_SIZE) & ((y2_c - y1_c) >= _MIN_SIZE)
               & (score_c >= _SCORE_THRESH) & (i_c < _N))
    s_c = score_c * valid_c.astype(f32)

    xx1 = jnp.maximum(x1_c, x1_r)
    yy1 = jnp.maximum(y1_c, y1_r)
    xx2 = jnp.minimum(x2_c, x2_r)
    yy2 = jnp.minimum(y2_c, y2_r)
    iw = jnp.clip(xx2 - xx1, 0.0)
    ih = jnp.clip(yy2 - yy1, 0.0)
    inter = iw * ih
    area_c = (x2_c - x1_c) * (y2_c - y1_c)
    area_r = (x2_r - x1_r) * (y2_r - y1_r)
    iou = inter / (area_c + area_r - inter + 1e-9)
    ov = iou > _NMS_THRESH

    prec = (s_c > s_r) | ((s_c == s_r) & (i_c < j_r))
    sup = (ov & prec).astype(f32)
    prec_t = (s_r > s_c) | ((s_r == s_c) & (j_r < i_c))
    sup_t = (ov & prec_t).astype(f32)

    def cond(st):
        return st[2]

    def body(st):
        k_r, k_c, _ = st
        t_r = jnp.sum(sup * k_c, axis=0, keepdims=True)
        t_c = jnp.sum(sup_t * k_r, axis=1, keepdims=True)
        nk_r = (t_r == 0.0).astype(f32)
        nk_c = (t_c == 0.0).astype(f32)
        return nk_r, nk_c, jnp.any(nk_r != k_r)

    k_r, k_c, _ = lax.while_loop(
        cond, body,
        (jnp.ones((1, _NP), f32), jnp.ones((_NP, 1), f32),
         jnp.asarray(True)))

    eff_r = s_r * k_r
    eff_c = s_c * k_c

    before = ((eff_c > eff_r)
              | ((eff_c == eff_r)
                 & ((s_c > s_r) | ((s_c == s_r) & (i_c < j_r)))))
    rank_r = jnp.sum(before.astype(f32), axis=0, keepdims=True)
    rank_i = rank_r.astype(jnp.int32)

    p_iota = lax.broadcasted_iota(jnp.int32, (_OUTP, _NP), 0)
    onehot = (rank_i == p_iota).astype(f32)
    pieces = []
    for v in (x1_r, y1_r, x2_r, y2_r, eff_r):
        pieces.append(jnp.sum(onehot * v, axis=1, keepdims=True))
    zero = jnp.zeros((_OUTP, 1), f32)
    out_ref[0, :, :] = jnp.concatenate(pieces + [zero, zero, zero], axis=1)


def kernel(class_logit, box_regression, proposal, image_w, image_h):
    f32 = jnp.float32
    lg = class_logit.astype(f32)
    lg_c = jnp.pad(lg, ((0, _NP - _N), (0, _CP - _C)), constant_values=_NEG)
    lg_t = lg_c.T

    br = box_regression.astype(f32).reshape(_N, _C, 4)
    dl_t = jnp.pad(jnp.transpose(br, (1, 2, 0)), ((0, 0), (0, 0), (0, _NP - _N)))
    dl_c = jnp.pad(jnp.transpose(br, (1, 0, 2)), ((0, 0), (0, _NP - _N), (0, 0)))

    pr = proposal.astype(f32)
    pr_t = jnp.pad(pr.T, ((0, 0), (0, _NP - _N)))
    pr_c = jnp.pad(pr, ((0, _NP - _N), (0, 0)))

    wh = jnp.stack([jnp.asarray(image_w, f32), jnp.asarray(image_h, f32)])

    out = pl.pallas_call(
        _nms_body,
        grid=(_C - 1,),
        in_specs=[
            pl.BlockSpec(memory_space=pltpu.SMEM),
            pl.BlockSpec((_CP, _NP), lambda c: (0, 0)),
            pl.BlockSpec((_NP, _CP), lambda c: (0, 0)),
            pl.BlockSpec((1, 4, _NP), lambda c: (c + 1, 0, 0)),
            pl.BlockSpec((1, _NP, 4), lambda c: (c + 1, 0, 0)),
            pl.BlockSpec((4, _NP), lambda c: (0, 0)),
            pl.BlockSpec((_NP, 4), lambda c: (0, 0)),
        ],
        out_specs=pl.BlockSpec((1, _OUTP, 8), lambda c: (c, 0, 0)),
        out_shape=jax.ShapeDtypeStruct((_C - 1, _OUTP, 8), f32),
    )(wh, lg_t, lg_c, dl_t, dl_c, pr_t, pr_c)
    return out[:, :_NUM_DETECTIONS, :5]
